# baseline (speedup 1.0000x reference)
import jax
import jax.numpy as jnp
from jax.experimental import pallas as pl
from jax.experimental.pallas import tpu as pltpu

N = 4096
D = 256
BM = 512
NSTEPS = N // BM
NBUF = 4


def _gemm_pipeline(adj_hbm, emb_ref, out_hbm, adj_buf, out_buf, in_sems,
                   out_sems):
    for i in range(NBUF):
        pltpu.make_async_copy(
            adj_hbm.at[pl.ds(i * BM, BM), :], adj_buf.at[i], in_sems.at[i]
        ).start()
    for step in range(NSTEPS):
        slot = step % NBUF
        oslot = step % 2
        pltpu.make_async_copy(
            adj_hbm.at[pl.ds(step * BM, BM), :], adj_buf.at[slot],
            in_sems.at[slot]
        ).wait()
        if step >= 2:
            pltpu.make_async_copy(
                out_buf.at[oslot], out_hbm.at[pl.ds((step - 2) * BM, BM), :],
                out_sems.at[oslot]
            ).wait()
        out_buf[oslot] = jax.lax.dot_general(
            adj_buf[slot], emb_ref[...],
            dimension_numbers=(((1,), (0,)), ((), ())),
            precision=jax.lax.Precision.DEFAULT,
            preferred_element_type=jnp.float32,
        )
        pltpu.make_async_copy(
            out_buf.at[oslot], out_hbm.at[pl.ds(step * BM, BM), :],
            out_sems.at[oslot]
        ).start()
        nxt = step + NBUF
        if nxt < NSTEPS:
            pltpu.make_async_copy(
                adj_hbm.at[pl.ds(nxt * BM, BM), :], adj_buf.at[slot],
                in_sems.at[slot]
            ).start()
    for step in range(NSTEPS - 2, NSTEPS):
        pltpu.make_async_copy(
            out_buf.at[step % 2], out_hbm.at[pl.ds(step * BM, BM), :],
            out_sems.at[step % 2]
        ).wait()


def kernel(adj, embeds):
    return pl.pallas_call(
        _gemm_pipeline,
        in_specs=[
            pl.BlockSpec(memory_space=pl.ANY),
            pl.BlockSpec(memory_space=pltpu.VMEM),
        ],
        out_specs=pl.BlockSpec(memory_space=pl.ANY),
        out_shape=jax.ShapeDtypeStruct((N, D), jnp.float32),
        scratch_shapes=[
            pltpu.VMEM((NBUF, BM, N), jnp.float32),
            pltpu.VMEM((2, BM, D), jnp.float32),
            pltpu.SemaphoreType.DMA((NBUF,)),
            pltpu.SemaphoreType.DMA((2,)),
        ],
    )(adj, embeds)


# speedup vs baseline: 1.0995x; 1.0995x over previous
import functools

import jax
import jax.numpy as jnp
from jax.experimental import pallas as pl
from jax.experimental.pallas import tpu as pltpu

N = 4096
D = 256
BM = 512


def _matmul_block(adj_ref, emb_ref, out_ref):
    out_ref[...] = jax.lax.dot_general(
        adj_ref[...], emb_ref[...],
        dimension_numbers=(((1,), (0,)), ((), ())),
        precision=jax.lax.Precision.DEFAULT,
        preferred_element_type=jnp.float32,
    )


@functools.partial(jax.jit, static_argnames=())
def kernel(adj, embeds):
    return pl.pallas_call(
        _matmul_block,
        grid=(N // BM,),
        in_specs=[
            pl.BlockSpec((BM, N), lambda i: (i, 0)),
            pl.BlockSpec((N, D), lambda i: (0, 0)),
        ],
        out_specs=pl.BlockSpec((BM, D), lambda i: (i, 0)),
        out_shape=jax.ShapeDtypeStruct((N, D), jnp.float32),
        compiler_params=pltpu.CompilerParams(
            dimension_semantics=("parallel",),
        ),
    )(adj, embeds)
